# baseline (speedup 1.0000x reference)
import functools

import jax
import jax.numpy as jnp
from jax import lax
from jax.experimental import pallas as pl
from jax.experimental.pallas import tpu as pltpu
from jax.experimental.pallas import tpu_sc as plsc

B = 16
S = 4096
HALF = S // 2
LANES = 16
CHUNKS = HALF // LANES
ACCS = 8


def _make_sc_kernel():
  mesh = plsc.VectorSubcoreMesh(core_axis_name="c", subcore_axis_name="s")

  @functools.partial(
      pl.kernel,
      mesh=mesh,
      out_type=jax.ShapeDtypeStruct((B * S,), jnp.float32),
      scratch_types=[
          pltpu.VMEM((HALF,), jnp.float32),
          pltpu.VMEM((HALF,), jnp.float32),
          pltpu.SemaphoreType.DMA,
          pltpu.SemaphoreType.DMA,
      ],
      compiler_params=pltpu.CompilerParams(needs_layout_passes=False),
  )
  def cumsum_kernel(pad_hbm, out_hbm, buf_pre, buf, sem_pre, sem_own):
    cid = lax.axis_index("c")
    sid = lax.axis_index("s")
    wid = sid * 2 + cid
    row = wid // 2
    half = wid % 2

    row_base = row * S
    own_off = row_base + half * HALF

    cp_pre = pltpu.async_copy(pad_hbm.at[pl.ds(row_base, HALF)], buf_pre,
                              sem_pre)
    cp_own = pltpu.async_copy(pad_hbm.at[pl.ds(own_off, HALF)], buf, sem_own)
    cp_pre.wait()

    def acc_body(g, accs):
      return tuple(
          accs[k] + buf_pre[pl.ds((g * ACCS + k) * LANES, LANES)]
          for k in range(ACCS)
      )

    accs = lax.fori_loop(
        0, CHUNKS // ACCS, acc_body,
        tuple(jnp.zeros((LANES,), jnp.float32) for _ in range(ACCS)))
    acc = accs[0]
    for k in range(1, ACCS):
      acc = acc + accs[k]
    prefix_total = jnp.sum(acc) * half.astype(jnp.float32)

    cp_own.wait()

    lane15 = jnp.full((LANES,), LANES - 1, jnp.int32)

    def scan_body(i, carry):
      base = i * LANES
      v = buf[pl.ds(base, LANES)]
      buf[pl.ds(base, LANES)] = plsc.cumsum(v) + carry
      return plsc.load_gather(buf, [lane15 + base])

    lax.fori_loop(0, CHUNKS,
                  scan_body,
                  jnp.full((LANES,), prefix_total - 1.0, jnp.float32),
                  unroll=8)

    pltpu.sync_copy(buf, out_hbm.at[pl.ds(own_off, HALF)])

  return cumsum_kernel


_sc_cumsum = _make_sc_kernel()


@jax.jit
def kernel(pad_masks):
  flat = pad_masks.reshape(-1)
  out = _sc_cumsum(flat)
  return out.reshape(B, S)


# speedup vs baseline: 1.0481x; 1.0481x over previous
import functools

import jax
import jax.numpy as jnp
from jax import lax
from jax.experimental import pallas as pl
from jax.experimental.pallas import tpu as pltpu
from jax.experimental.pallas import tpu_sc as plsc

B = 16
S = 4096
LANES = 16
CHUNKS = S // LANES


def _make_sc_kernel():
  mesh = plsc.VectorSubcoreMesh(
      core_axis_name="c", subcore_axis_name="s", num_cores=1)

  @functools.partial(
      pl.kernel,
      mesh=mesh,
      out_type=jax.ShapeDtypeStruct((B * S,), jnp.float32),
      scratch_types=[
          pltpu.VMEM((S,), jnp.float32),
          pltpu.SemaphoreType.DMA,
      ],
      compiler_params=pltpu.CompilerParams(needs_layout_passes=False),
  )
  def cumsum_kernel(pad_hbm, out_hbm, buf, sem):
    row_base = lax.axis_index("s") * S

    pltpu.async_copy(pad_hbm.at[pl.ds(row_base, S)], buf, sem).wait()

    lane15 = jnp.full((LANES,), LANES - 1, jnp.int32)

    def scan_body(i, carry):
      base = i * LANES
      v = buf[pl.ds(base, LANES)]
      buf[pl.ds(base, LANES)] = plsc.cumsum(v) + carry
      return plsc.load_gather(buf, [lane15 + base])

    lax.fori_loop(0, CHUNKS, scan_body,
                  jnp.full((LANES,), -1.0, jnp.float32), unroll=4)

    pltpu.sync_copy(buf, out_hbm.at[pl.ds(row_base, S)])

  return cumsum_kernel


_sc_cumsum = _make_sc_kernel()


@jax.jit
def kernel(pad_masks):
  flat = pad_masks.reshape(-1)
  out = _sc_cumsum(flat)
  return out.reshape(B, S)


# speedup vs baseline: 1.1440x; 1.0915x over previous
import functools

import jax
import jax.numpy as jnp
from jax import lax
from jax.experimental import pallas as pl
from jax.experimental.pallas import tpu as pltpu
from jax.experimental.pallas import tpu_sc as plsc

B = 16
S = 4096
LANES = 16
CHUNKS = S // LANES


def _make_sc_kernel():
  mesh = plsc.VectorSubcoreMesh(
      core_axis_name="c", subcore_axis_name="s", num_cores=1)

  @functools.partial(
      pl.kernel,
      mesh=mesh,
      out_type=jax.ShapeDtypeStruct((B, S), jnp.float32),
      scratch_types=[
          pltpu.VMEM((S,), jnp.float32),
          pltpu.SemaphoreType.DMA,
      ],
      compiler_params=pltpu.CompilerParams(needs_layout_passes=False),
  )
  def cumsum_kernel(pad_hbm, out_hbm, buf, sem):
    row = lax.axis_index("s")

    pltpu.async_copy(pad_hbm.at[row], buf, sem).wait()

    lane15 = jnp.full((LANES,), LANES - 1, jnp.int32)

    def scan_body(i, carry):
      base = i * LANES
      v = buf[pl.ds(base, LANES)]
      buf[pl.ds(base, LANES)] = plsc.cumsum(v) + carry
      return plsc.load_gather(buf, [lane15 + base])

    lax.fori_loop(0, CHUNKS, scan_body,
                  jnp.full((LANES,), -1.0, jnp.float32), unroll=4)

    pltpu.sync_copy(buf, out_hbm.at[row])

  return cumsum_kernel


_sc_cumsum = _make_sc_kernel()


@jax.jit
def kernel(pad_masks):
  return _sc_cumsum(pad_masks)
